# baseline (speedup 1.0000x reference)
import functools

import jax
import jax.numpy as jnp
from jax import lax
from jax.experimental import pallas as pl
from jax.experimental.pallas import tpu as pltpu
from jax.experimental.pallas import tpu_sc as plsc

N = 10000
E = 320000
D = 128
H = 2 * D
BN_EPS = 1e-5

NC = 2
NS = 16
NW = NC * NS

N_PAD = 10112
ROWS_PER_TILE = N_PAD // NS
CHUNK = 80
TOTAL_CHUNKS = E // CHUNK
BASE_CHUNKS = TOTAL_CHUNKS // NW
EXTRA_CHUNKS = TOTAL_CHUNKS - BASE_CHUNKS * NW


SUP = 25
NSUP = 125 // SUP
SUP_E = SUP * CHUNK


def _sc_body(x_hbm, src_hbm, dst_hbm, ea_hbm, zero_hbm, out_hbm,
             srcS, dstS, src_v, dst_v, ea0, ea1, xr0, xr1, acc_sh,
             sem_ea, sem_gx, sem_stage):
    c = lax.axis_index("c")
    s = lax.axis_index("s")
    wid = s * jnp.int32(NC) + c

    row0 = s * jnp.int32(ROWS_PER_TILE)
    pltpu.sync_copy(zero_hbm, acc_sh.at[pl.ds(row0, ROWS_PER_TILE)])
    plsc.subcore_barrier()

    base_e = wid * jnp.int32(125 * CHUNK)
    nchunks = jnp.int32(125)

    def issue_stage(sup):
        off = base_e + sup * jnp.int32(SUP_E)
        sb = (sup % jnp.int32(2)) * jnp.int32(SUP_E)
        pltpu.async_copy(src_hbm.at[pl.ds(off, SUP_E)],
                         srcS.at[pl.ds(sb, SUP_E)], sem_stage)
        pltpu.async_copy(dst_hbm.at[pl.ds(off, SUP_E)],
                         dstS.at[pl.ds(sb, SUP_E)], sem_stage)

    def wait_stage():
        pltpu.make_async_copy(src_hbm.at[pl.ds(0, SUP_E)],
                              srcS.at[pl.ds(0, SUP_E)], sem_stage).wait()
        pltpu.make_async_copy(dst_hbm.at[pl.ds(0, SUP_E)],
                              dstS.at[pl.ds(0, SUP_E)], sem_stage).wait()

    def stage_off(g):
        sb = ((g // jnp.int32(SUP)) % jnp.int32(2)) * jnp.int32(SUP_E)
        return sb + (g % jnp.int32(SUP)) * jnp.int32(CHUNK)

    def issue_io(g, ea_v, xr_v):
        pltpu.async_copy(ea_hbm.at[pl.ds(base_e + g * jnp.int32(CHUNK), CHUNK)],
                         ea_v, sem_ea)
        ioff = stage_off(g)
        for k in range(CHUNK // 16):
            src_v[pl.ds(k * 16, 16)] = srcS[pl.ds(ioff + k * 16, 16)]
        pltpu.async_copy(x_hbm.at[src_v], xr_v, sem_gx)

    def wait_io(ea_v, xr_v):
        pltpu.make_async_copy(ea_hbm.at[pl.ds(0, CHUNK)], ea_v, sem_ea).wait()
        pltpu.make_async_copy(x_hbm.at[pl.ds(0, CHUNK)], xr_v, sem_gx).wait()

    def do_chunk(g, ea_c, xr_c, ea_n, xr_n):
        wait_io(ea_c, xr_c)

        @pl.when(g % jnp.int32(SUP) == jnp.int32(0))
        def _():
            @pl.when(g + jnp.int32(SUP) < nchunks)
            def _():
                issue_stage(g // jnp.int32(SUP) + jnp.int32(1))

        @pl.when((g + jnp.int32(1)) % jnp.int32(SUP) == jnp.int32(0))
        def _():
            @pl.when(g + jnp.int32(1) < nchunks)
            def _():
                wait_stage()

        @pl.when(g + jnp.int32(1) < nchunks)
        def _():
            issue_io(g + jnp.int32(1), ea_n, xr_n)

        def row_body(r, cc2):
            for k in range(D // 16):
                sl = pl.ds(k * 16, 16)
                xr_c[r, sl] = jnp.maximum(xr_c[r, sl] + ea_c[r, sl], 0.0)
            return cc2

        lax.fori_loop(jnp.int32(0), jnp.int32(CHUNK), row_body, jnp.int32(0))

        ioff = stage_off(g)
        for k in range(CHUNK // 16):
            dst_v[pl.ds(k * 16, 16)] = dstS[pl.ds(ioff + k * 16, 16)]

        pltpu.sync_copy(xr_c, acc_sh.at[dst_v], add=True)

    issue_stage(jnp.int32(0))
    wait_stage()
    issue_io(jnp.int32(0), ea0, xr0)

    def pair_body(p, cc):
        g = p * jnp.int32(2)
        do_chunk(g, ea0, xr0, ea1, xr1)
        do_chunk(g + jnp.int32(1), ea1, xr1, ea0, xr0)
        return cc

    lax.fori_loop(jnp.int32(0), jnp.int32(62), pair_body, jnp.int32(0))
    do_chunk(jnp.int32(124), ea0, xr0, ea1, xr1)

    plsc.subcore_barrier()
    rs = pl.ds(row0, ROWS_PER_TILE)
    pltpu.sync_copy(acc_sh.at[rs], out_hbm.at[c, rs])


@functools.cache
def _sc_scatter():
    return pl.kernel(
        _sc_body,
        mesh=plsc.VectorSubcoreMesh(core_axis_name="c", subcore_axis_name="s"),
        out_type=jax.ShapeDtypeStruct((NC, N_PAD, D), jnp.float32),
        scratch_types=[
            pltpu.VMEM((2 * SUP_E,), jnp.int32),
            pltpu.VMEM((2 * SUP_E,), jnp.int32),
            pltpu.VMEM((CHUNK,), jnp.int32),
            pltpu.VMEM((CHUNK,), jnp.int32),
            pltpu.VMEM((CHUNK, D), jnp.float32),
            pltpu.VMEM((CHUNK, D), jnp.float32),
            pltpu.VMEM((CHUNK, D), jnp.float32),
            pltpu.VMEM((CHUNK, D), jnp.float32),
            pltpu.VMEM_SHARED((N_PAD, D), jnp.float32),
            pltpu.SemaphoreType.DMA,
            pltpu.SemaphoreType.DMA,
            pltpu.SemaphoreType.DMA,
        ],
    )


def _tc_body(acc_ref, x_ref, w1_ref, g1_ref, b1_ref, w2_ref, g2_ref,
             b2_ref, eps_ref, o_ref):
    nn = acc_ref[0][:N, :] + acc_ref[1][:N, :]
    h = nn + (1.0 + eps_ref[0, 0]) * x_ref[...]
    h = jnp.dot(h, w1_ref[...], preferred_element_type=jnp.float32,
                precision=lax.Precision.HIGHEST)
    mu = jnp.mean(h, axis=0, keepdims=True)
    d = h - mu
    var = jnp.mean(d * d, axis=0, keepdims=True)
    h = d * lax.rsqrt(var + BN_EPS) * g1_ref[...] + b1_ref[...]
    h = jnp.maximum(h, 0.0)
    h = jnp.dot(h, w2_ref[...], preferred_element_type=jnp.float32,
                precision=lax.Precision.HIGHEST)
    mu = jnp.mean(h, axis=0, keepdims=True)
    d = h - mu
    var = jnp.mean(d * d, axis=0, keepdims=True)
    h = d * lax.rsqrt(var + BN_EPS) * g2_ref[...] + b2_ref[...]
    o_ref[...] = jnp.maximum(h, 0.0)


_tc_mlp = pl.pallas_call(
    _tc_body,
    out_shape=jax.ShapeDtypeStruct((N, D), jnp.float32),
)


@jax.jit
def kernel(x, edge_index, edge_attr, W1, gamma1, beta1, W2, gamma2, beta2,
           epsilon):
    out_dtype = jnp.result_type(x.dtype, W1.dtype, W2.dtype)
    src = edge_index[0].astype(jnp.int32)
    dst = edge_index[1].astype(jnp.int32)
    zero = jnp.zeros((ROWS_PER_TILE, D), jnp.float32)
    acc = _sc_scatter()(x, src, dst, edge_attr, zero)
    out = _tc_mlp(acc, x, W1.astype(jnp.float32),
                  gamma1.reshape(1, H).astype(jnp.float32),
                  beta1.reshape(1, H).astype(jnp.float32),
                  W2.astype(jnp.float32),
                  gamma2.reshape(1, D).astype(jnp.float32),
                  beta2.reshape(1, D).astype(jnp.float32),
                  epsilon.reshape(1, 1).astype(jnp.float32))
    return out.astype(out_dtype)


# speedup vs baseline: 1.0470x; 1.0470x over previous
import functools

import jax
import jax.numpy as jnp
from jax import lax
from jax.experimental import pallas as pl
from jax.experimental.pallas import tpu as pltpu
from jax.experimental.pallas import tpu_sc as plsc

N = 10000
E = 320000
D = 128
H = 2 * D
BN_EPS = 1e-5

NC = 2
NS = 16
NW = NC * NS

N_PAD = 10112
ROWS_PER_TILE = N_PAD // NS
CHUNK = 80
TOTAL_CHUNKS = E // CHUNK
BASE_CHUNKS = TOTAL_CHUNKS // NW
EXTRA_CHUNKS = TOTAL_CHUNKS - BASE_CHUNKS * NW


SUP = 25
NSUP = 125 // SUP
SUP_E = SUP * CHUNK


def _sc_body(x_hbm, src_hbm, dst_hbm, ea_hbm, zero_hbm, out_hbm,
             srcS, dstS, sv0, sv1, dst_v, ea0, ea1, xr0, xr1, acc_sh,
             sem_io0, sem_io1, sem_stage):
    c = lax.axis_index("c")
    s = lax.axis_index("s")
    wid = s * jnp.int32(NC) + c

    row0 = s * jnp.int32(ROWS_PER_TILE)
    pltpu.sync_copy(zero_hbm, acc_sh.at[pl.ds(row0, ROWS_PER_TILE)])
    plsc.subcore_barrier()

    base_e = wid * jnp.int32(125 * CHUNK)
    nchunks = jnp.int32(125)

    def issue_stage(sup):
        off = base_e + sup * jnp.int32(SUP_E)
        sb = (sup % jnp.int32(2)) * jnp.int32(SUP_E)
        pltpu.async_copy(src_hbm.at[pl.ds(off, SUP_E)],
                         srcS.at[pl.ds(sb, SUP_E)], sem_stage)
        pltpu.async_copy(dst_hbm.at[pl.ds(off, SUP_E)],
                         dstS.at[pl.ds(sb, SUP_E)], sem_stage)

    def wait_stage():
        pltpu.make_async_copy(src_hbm.at[pl.ds(0, SUP_E)],
                              srcS.at[pl.ds(0, SUP_E)], sem_stage).wait()
        pltpu.make_async_copy(dst_hbm.at[pl.ds(0, SUP_E)],
                              dstS.at[pl.ds(0, SUP_E)], sem_stage).wait()

    def stage_off(g):
        sb = ((g // jnp.int32(SUP)) % jnp.int32(2)) * jnp.int32(SUP_E)
        return sb + (g % jnp.int32(SUP)) * jnp.int32(CHUNK)

    def issue_io(g, P):
        src_v, ea_v, xr_v, sem = P
        pltpu.async_copy(ea_hbm.at[pl.ds(base_e + g * jnp.int32(CHUNK), CHUNK)],
                         ea_v, sem)
        ioff = stage_off(g)
        for k in range(CHUNK // 16):
            src_v[pl.ds(k * 16, 16)] = srcS[pl.ds(ioff + k * 16, 16)]
        pltpu.async_copy(x_hbm.at[src_v], xr_v, sem)

    def wait_io(P):
        _, ea_v, xr_v, sem = P
        pltpu.make_async_copy(ea_hbm.at[pl.ds(0, CHUNK)], ea_v, sem).wait()
        pltpu.make_async_copy(x_hbm.at[pl.ds(0, CHUNK)], xr_v, sem).wait()

    def do_chunk(g, cur, nxt):

        @pl.when(g % jnp.int32(SUP) == jnp.int32(0))
        def _():
            @pl.when(g + jnp.int32(SUP) < nchunks)
            def _():
                issue_stage(g // jnp.int32(SUP) + jnp.int32(1))

        @pl.when((g + jnp.int32(1)) % jnp.int32(SUP) == jnp.int32(0))
        def _():
            @pl.when(g + jnp.int32(1) < nchunks)
            def _():
                wait_stage()

        @pl.when(g + jnp.int32(1) < nchunks)
        def _():
            issue_io(g + jnp.int32(1), nxt)

        wait_io(cur)
        ea_c, xr_c = cur[1], cur[2]

        def row_body(r, cc2):
            for k in range(D // 16):
                sl = pl.ds(k * 16, 16)
                xr_c[r, sl] = jnp.maximum(xr_c[r, sl] + ea_c[r, sl], 0.0)
            return cc2

        lax.fori_loop(jnp.int32(0), jnp.int32(CHUNK), row_body, jnp.int32(0))

        ioff = stage_off(g)
        for k in range(CHUNK // 16):
            dst_v[pl.ds(k * 16, 16)] = dstS[pl.ds(ioff + k * 16, 16)]

        pltpu.sync_copy(xr_c, acc_sh.at[dst_v], add=True)

    P0 = (sv0, ea0, xr0, sem_io0)
    P1 = (sv1, ea1, xr1, sem_io1)

    issue_stage(jnp.int32(0))
    wait_stage()
    issue_io(jnp.int32(0), P0)

    def pair_body(p, cc):
        g = p * jnp.int32(2)
        do_chunk(g, P0, P1)
        do_chunk(g + jnp.int32(1), P1, P0)
        return cc

    lax.fori_loop(jnp.int32(0), jnp.int32(62), pair_body, jnp.int32(0))
    do_chunk(jnp.int32(124), P0, P1)

    plsc.subcore_barrier()
    rs = pl.ds(row0, ROWS_PER_TILE)
    pltpu.sync_copy(acc_sh.at[rs], out_hbm.at[c, rs])


@functools.cache
def _sc_scatter():
    return pl.kernel(
        _sc_body,
        mesh=plsc.VectorSubcoreMesh(core_axis_name="c", subcore_axis_name="s"),
        out_type=jax.ShapeDtypeStruct((NC, N_PAD, D), jnp.float32),
        scratch_types=[
            pltpu.VMEM((2 * SUP_E,), jnp.int32),
            pltpu.VMEM((2 * SUP_E,), jnp.int32),
            pltpu.VMEM((CHUNK,), jnp.int32),
            pltpu.VMEM((CHUNK,), jnp.int32),
            pltpu.VMEM((CHUNK,), jnp.int32),
            pltpu.VMEM((CHUNK, D), jnp.float32),
            pltpu.VMEM((CHUNK, D), jnp.float32),
            pltpu.VMEM((CHUNK, D), jnp.float32),
            pltpu.VMEM((CHUNK, D), jnp.float32),
            pltpu.VMEM_SHARED((N_PAD, D), jnp.float32),
            pltpu.SemaphoreType.DMA,
            pltpu.SemaphoreType.DMA,
            pltpu.SemaphoreType.DMA,
        ],
    )


def _tc_body(acc_ref, x_ref, w1_ref, g1_ref, b1_ref, w2_ref, g2_ref,
             b2_ref, eps_ref, o_ref):
    nn = acc_ref[0][:N, :] + acc_ref[1][:N, :]
    h = nn + (1.0 + eps_ref[0, 0]) * x_ref[...]
    h = jnp.dot(h, w1_ref[...], preferred_element_type=jnp.float32,
                precision=lax.Precision.HIGHEST)
    mu = jnp.mean(h, axis=0, keepdims=True)
    d = h - mu
    var = jnp.mean(d * d, axis=0, keepdims=True)
    h = d * lax.rsqrt(var + BN_EPS) * g1_ref[...] + b1_ref[...]
    h = jnp.maximum(h, 0.0)
    h = jnp.dot(h, w2_ref[...], preferred_element_type=jnp.float32,
                precision=lax.Precision.HIGHEST)
    mu = jnp.mean(h, axis=0, keepdims=True)
    d = h - mu
    var = jnp.mean(d * d, axis=0, keepdims=True)
    h = d * lax.rsqrt(var + BN_EPS) * g2_ref[...] + b2_ref[...]
    o_ref[...] = jnp.maximum(h, 0.0)


_tc_mlp = pl.pallas_call(
    _tc_body,
    out_shape=jax.ShapeDtypeStruct((N, D), jnp.float32),
)


@jax.jit
def kernel(x, edge_index, edge_attr, W1, gamma1, beta1, W2, gamma2, beta2,
           epsilon):
    out_dtype = jnp.result_type(x.dtype, W1.dtype, W2.dtype)
    src = edge_index[0].astype(jnp.int32)
    dst = edge_index[1].astype(jnp.int32)
    zero = jnp.zeros((ROWS_PER_TILE, D), jnp.float32)
    acc = _sc_scatter()(x, src, dst, edge_attr, zero)
    out = _tc_mlp(acc, x, W1.astype(jnp.float32),
                  gamma1.reshape(1, H).astype(jnp.float32),
                  beta1.reshape(1, H).astype(jnp.float32),
                  W2.astype(jnp.float32),
                  gamma2.reshape(1, D).astype(jnp.float32),
                  beta2.reshape(1, D).astype(jnp.float32),
                  epsilon.reshape(1, 1).astype(jnp.float32))
    return out.astype(out_dtype)
